# baseline (speedup 1.0000x reference)
import functools

import jax
import jax.numpy as jnp
from jax import lax
from jax.experimental import pallas as pl
from jax.experimental.pallas import tpu as pltpu
from jax.experimental.pallas import tpu_sc as plsc

B, T, H = 1024, 50, 96
E = 32
G4 = 4 * H
FE = 4 * E

NC, NS = 2, 16
NW = NC * NS
BT = B * T
RPW = BT // NW
CH = 80
NCH = RPW // CH

BB = 256
NB = B // BB


def _sc_gather_body(idx_hbm, t_inter, t_test, t_q, t_tag, out_hbm,
                    idx_v, rows_v, gsem, osem):
    wid = lax.axis_index("s") * NC + lax.axis_index("c")
    base = wid * RPW
    tables = (t_inter, t_test, t_q, t_tag)
    pltpu.sync_copy(idx_hbm.at[wid], idx_v)
    out_copies = [None, None]
    gather_waves = []
    for j, tab in enumerate(tables):
        s = j % 2
        if out_copies[s] is not None:
            out_copies[s].wait()
        copies = []
        for c in range(NCH):
            copies.append(
                pltpu.async_copy(tab.at[idx_v.at[j, c]],
                                 rows_v.at[s, pl.ds(c * CH, CH)], gsem))
        gather_waves.append(copies)
        if j >= 1:
            for cp in gather_waves[j - 1]:
                cp.wait()
            out_copies[(j - 1) % 2] = pltpu.async_copy(
                rows_v.at[(j - 1) % 2],
                out_hbm.at[j - 1, pl.ds(base, RPW)], osem)
    for cp in gather_waves[3]:
        cp.wait()
    out_copies[1] = pltpu.async_copy(
        rows_v.at[1], out_hbm.at[3, pl.ds(base, RPW)], osem)
    out_copies[0].wait()
    out_copies[1].wait()


@functools.partial(jax.jit, static_argnums=())
def _sc_gather(idx, emb_inter, emb_test, emb_q, emb_tag):
    mesh = plsc.VectorSubcoreMesh(core_axis_name="c", subcore_axis_name="s")
    return pl.kernel(
        _sc_gather_body,
        out_type=jax.ShapeDtypeStruct((4, BT, E), jnp.float32),
        mesh=mesh,
        compiler_params=pltpu.CompilerParams(use_tc_tiling_on_sc=False),
        scratch_types=[
            pltpu.VMEM((4, NCH, CH), jnp.int32),
            pltpu.VMEM((2, RPW, E), jnp.float32),
            pltpu.SemaphoreType.DMA,
            pltpu.SemaphoreType.DMA,
        ],
    )(idx, emb_inter, emb_test, emb_q, emb_tag)


BB4 = BB // 4


def _tc_body(e_ref, Wt_ref, bc_ref, Wih0_ref, Whh0_ref, b0_ref,
             Wih1_ref, Whh1_ref, b1_ref, Wf_ref, bf_ref,
             out_ref, Xg_ref, hseq_ref):
    cdims = (((1,), (1,)), ((), ()))

    em = e_ref[...]
    Wt = Wt_ref[...]
    Xp = lax.dot_general(em[0].reshape(T * BB4, FE), Wt[0], cdims,
                         preferred_element_type=jnp.float32)
    for j in range(1, 4):
        Xp += lax.dot_general(em[j].reshape(T * BB4, FE), Wt[j], cdims,
                              preferred_element_type=jnp.float32)
    for k in range(4):
        Xk = Xp[:, k * H:(k + 1) * H] + bc_ref[...]
        Xg_ref[:, k] = (lax.dot_general(Xk, Wih0_ref[...], cdims,
                                        preferred_element_type=jnp.float32)
                        + b0_ref[...]).reshape(T, BB4, G4)

    def recur(Whh_ref):
        def step(t, carry):
            h, c = carry
            g = Xg_ref[t].reshape(BB, G4) + lax.dot_general(
                h, Whh_ref[...], cdims, preferred_element_type=jnp.float32)
            i = jax.nn.sigmoid(g[:, 0:H])
            f = jax.nn.sigmoid(g[:, H:2 * H])
            gg = jnp.tanh(g[:, 2 * H:3 * H])
            o = jax.nn.sigmoid(g[:, 3 * H:4 * H])
            c = f * c + i * gg
            h = o * jnp.tanh(c)
            hseq_ref[t] = h
            return (h, c)
        z = jnp.zeros((BB, H), jnp.float32)
        lax.fori_loop(0, T, step, (z, z))

    recur(Whh0_ref)
    Xg_ref[...] = (lax.dot_general(hseq_ref[...].reshape(T * BB, H),
                                   Wih1_ref[...], cdims,
                                   preferred_element_type=jnp.float32)
                   + b1_ref[...]).reshape(T, 4, BB4, G4)
    recur(Whh1_ref)
    out_ref[...] = (jnp.sum(hseq_ref[...] * Wf_ref[...][0], axis=-1)
                    + bf_ref[0, 0])


def _tc_lstm(e_tm, Wt, bc, Wih0, Whh0, b0, Wih1, Whh1, b1, Wf, bf,
             interpret=False):
    full = lambda shape: pl.BlockSpec(shape, lambda i: (0,) * len(shape))
    return pl.pallas_call(
        _tc_body,
        grid=(NB,),
        in_specs=[
            pl.BlockSpec((4, T, BB4, FE), lambda i: (0, 0, i, 0)),
            full((4, 4 * H, FE)), full((1, H)),
            full((G4, H)), full((G4, H)), full((1, G4)),
            full((G4, H)), full((G4, H)), full((1, G4)),
            full((1, H)), full((1, 1)),
        ],
        out_specs=pl.BlockSpec((T, BB), lambda i: (0, i)),
        out_shape=jax.ShapeDtypeStruct((T, B), jnp.float32),
        scratch_shapes=[
            pltpu.VMEM((T, 4, BB4, G4), jnp.float32),
            pltpu.VMEM((T, BB, H), jnp.float32),
        ],
        interpret=interpret,
    )(e_tm, Wt, bc, Wih0, Whh0, b0, Wih1, Whh1, b1, Wf, bf)


def kernel(test, question, tag, correct, mask, interaction, duration,
           emb_inter, emb_test, emb_q, emb_tag, Wc, bc,
           Wih0, Whh0, bih0, bhh0, Wih1, Whh1, bih1, bhh1, Wf, bf):
    idx = jnp.stack([
        interaction.T.reshape(-1), test.T.reshape(-1),
        question.T.reshape(-1), tag.T.reshape(-1),
    ]).reshape(4, NW, NCH, CH).transpose(1, 0, 2, 3)
    e = _sc_gather(idx, emb_inter, emb_test, emb_q, emb_tag)
    e_tm = e.reshape(4, T, B // 4, FE)
    Wt = jnp.stack([
        jax.scipy.linalg.block_diag(*([Wc[:, j * E:(j + 1) * E]] * 4))
        for j in range(4)
    ])
    out_tm = _tc_lstm(
        e_tm, Wt, bc.reshape(1, H),
        Wih0, Whh0, (bih0 + bhh0).reshape(1, G4),
        Wih1, Whh1, (bih1 + bhh1).reshape(1, G4),
        Wf, bf.reshape(1, 1))
    out_tm = out_tm.reshape(T, NB, 4, BB4).transpose(0, 1, 3, 2)
    return out_tm.reshape(T, B).T


# speedup vs baseline: 1.2885x; 1.2885x over previous
import functools

import jax
import jax.numpy as jnp
from jax import lax
from jax.experimental import pallas as pl
from jax.experimental.pallas import tpu as pltpu
from jax.experimental.pallas import tpu_sc as plsc

B, T, H = 1024, 50, 96
E = 32
G4 = 4 * H
FE = 4 * E

NBP = 4
BB = B // NBP
BB4 = BB // 4

NC, NS = 2, 16
NW = NC * NS
BT = BB * T
RPW = BT // NW
CH = 80
NCH = RPW // CH


def _sc_gather_body(idx_hbm, t_inter, t_test, t_q, t_tag, out_hbm,
                    idx_v, rows_v, gsem, osem):
    wid = lax.axis_index("s") * NC + lax.axis_index("c")
    base = wid * RPW
    tables = (t_inter, t_test, t_q, t_tag)
    pltpu.sync_copy(idx_hbm.at[wid], idx_v)
    out_copies = [None, None]
    gather_waves = []
    for j, tab in enumerate(tables):
        s = j % 2
        if out_copies[s] is not None:
            out_copies[s].wait()
        copies = []
        for c in range(NCH):
            copies.append(
                pltpu.async_copy(tab.at[idx_v.at[j, c]],
                                 rows_v.at[s, pl.ds(c * CH, CH)], gsem))
        gather_waves.append(copies)
        if j >= 1:
            for cp in gather_waves[j - 1]:
                cp.wait()
            out_copies[(j - 1) % 2] = pltpu.async_copy(
                rows_v.at[(j - 1) % 2],
                out_hbm.at[j - 1, pl.ds(base, RPW)], osem)
    for cp in gather_waves[3]:
        cp.wait()
    out_copies[1] = pltpu.async_copy(
        rows_v.at[1], out_hbm.at[3, pl.ds(base, RPW)], osem)
    out_copies[0].wait()
    out_copies[1].wait()


@functools.partial(jax.jit, static_argnums=())
def _sc_gather(idx, emb_inter, emb_test, emb_q, emb_tag):
    mesh = plsc.VectorSubcoreMesh(core_axis_name="c", subcore_axis_name="s")
    return pl.kernel(
        _sc_gather_body,
        out_type=jax.ShapeDtypeStruct((4, BT, E), jnp.float32),
        mesh=mesh,
        compiler_params=pltpu.CompilerParams(use_tc_tiling_on_sc=False),
        scratch_types=[
            pltpu.VMEM((4, NCH, CH), jnp.int32),
            pltpu.VMEM((2, RPW, E), jnp.float32),
            pltpu.SemaphoreType.DMA,
            pltpu.SemaphoreType.DMA,
        ],
    )(idx, emb_inter, emb_test, emb_q, emb_tag)


def _tc_body(e_ref, Wt_ref, bc_ref, Wih0_ref, Whh0_ref, b0_ref,
             Wih1_ref, Whh1_ref, b1_ref, Wf_ref, bf_ref,
             out_ref, Xg_ref, hseq_ref):
    cdims = (((1,), (1,)), ((), ()))

    em = e_ref[...]
    Wt = Wt_ref[...]
    Xp = lax.dot_general(em[0].reshape(T * BB4, FE), Wt[0], cdims,
                         preferred_element_type=jnp.float32)
    for j in range(1, 4):
        Xp += lax.dot_general(em[j].reshape(T * BB4, FE), Wt[j], cdims,
                              preferred_element_type=jnp.float32)
    for k in range(4):
        Xk = Xp[:, k * H:(k + 1) * H] + bc_ref[...]
        Xg_ref[:, k] = (lax.dot_general(Xk, Wih0_ref[...], cdims,
                                        preferred_element_type=jnp.float32)
                        + b0_ref[...]).reshape(T, BB4, G4)

    def recur(Whh_ref):
        def step(t, carry):
            h, c = carry
            g = Xg_ref[t].reshape(BB, G4) + lax.dot_general(
                h, Whh_ref[...], cdims, preferred_element_type=jnp.float32)
            i = jax.nn.sigmoid(g[:, 0:H])
            f = jax.nn.sigmoid(g[:, H:2 * H])
            gg = jnp.tanh(g[:, 2 * H:3 * H])
            o = jax.nn.sigmoid(g[:, 3 * H:4 * H])
            c = f * c + i * gg
            h = o * jnp.tanh(c)
            hseq_ref[t] = h
            return (h, c)
        z = jnp.zeros((BB, H), jnp.float32)
        lax.fori_loop(0, T, step, (z, z))

    recur(Whh0_ref)
    Xg_ref[...] = (lax.dot_general(hseq_ref[...].reshape(T * BB, H),
                                   Wih1_ref[...], cdims,
                                   preferred_element_type=jnp.float32)
                   + b1_ref[...]).reshape(T, 4, BB4, G4)
    recur(Whh1_ref)
    out_ref[...] = (jnp.sum(hseq_ref[...] * Wf_ref[...][0], axis=-1)
                    + bf_ref[0, 0])


def _tc_lstm(e_tm, Wt, bc, Wih0, Whh0, b0, Wih1, Whh1, b1, Wf, bf,
             interpret=False):
    return pl.pallas_call(
        _tc_body,
        out_shape=jax.ShapeDtypeStruct((T, BB), jnp.float32),
        scratch_shapes=[
            pltpu.VMEM((T, 4, BB4, G4), jnp.float32),
            pltpu.VMEM((T, BB, H), jnp.float32),
        ],
        interpret=interpret,
    )(e_tm, Wt, bc, Wih0, Whh0, b0, Wih1, Whh1, b1, Wf, bf)


def kernel(test, question, tag, correct, mask, interaction, duration,
           emb_inter, emb_test, emb_q, emb_tag, Wc, bc,
           Wih0, Whh0, bih0, bhh0, Wih1, Whh1, bih1, bhh1, Wf, bf):
    Wt = jnp.stack([
        jax.scipy.linalg.block_diag(*([Wc[:, j * E:(j + 1) * E]] * 4))
        for j in range(4)
    ])
    bc_r = bc.reshape(1, H)
    b0 = (bih0 + bhh0).reshape(1, G4)
    b1 = (bih1 + bhh1).reshape(1, G4)
    bf_r = bf.reshape(1, 1)
    outs = []
    for i in range(NBP):
        sl = slice(i * BB, (i + 1) * BB)
        idx = jnp.stack([
            interaction[sl].T.reshape(-1), test[sl].T.reshape(-1),
            question[sl].T.reshape(-1), tag[sl].T.reshape(-1),
        ]).reshape(4, NW, NCH, CH).transpose(1, 0, 2, 3)
        e = _sc_gather(idx, emb_inter, emb_test, emb_q, emb_tag)
        e_tm = e.reshape(4, T, BB4, FE)
        out_i = _tc_lstm(e_tm, Wt, bc_r, Wih0, Whh0, b0,
                         Wih1, Whh1, b1, Wf, bf_r)
        outs.append(out_i.reshape(T, 4, BB4).transpose(0, 2, 1))
    out_tm = jnp.concatenate(outs, axis=1).reshape(T, B)
    return out_tm.T


# speedup vs baseline: 1.9751x; 1.5328x over previous
import functools

import jax
import jax.numpy as jnp
from jax import lax
from jax.experimental import pallas as pl
from jax.experimental.pallas import tpu as pltpu
from jax.experimental.pallas import tpu_sc as plsc

B, T, H = 1024, 50, 96
E = 32
G4 = 4 * H
FE = 4 * E

NBP = 4
BB = B // NBP
BB4 = BB // 4

NT = 3
NC, NS = 2, 16
NW = NC * NS
BT = BB * T
RPW = BT // NW
CH = 80
NCH = RPW // CH


def _sc_gather_body(idx_hbm, t_test, t_q, t_tag, out_hbm,
                    idx_v, rows_v, gsem, osem):
    wid = lax.axis_index("s") * NC + lax.axis_index("c")
    base = wid * RPW
    tables = (t_test, t_q, t_tag)
    pltpu.sync_copy(idx_hbm.at[wid], idx_v)
    out_copies = [None, None]
    gather_waves = []
    for j, tab in enumerate(tables):
        s = j % 2
        if out_copies[s] is not None:
            out_copies[s].wait()
        copies = []
        for c in range(NCH):
            copies.append(
                pltpu.async_copy(tab.at[idx_v.at[j, c]],
                                 rows_v.at[s, pl.ds(c * CH, CH)], gsem))
        gather_waves.append(copies)
        if j >= 1:
            for cp in gather_waves[j - 1]:
                cp.wait()
            out_copies[(j - 1) % 2] = pltpu.async_copy(
                rows_v.at[(j - 1) % 2],
                out_hbm.at[j - 1, pl.ds(base, RPW)], osem)
    for cp in gather_waves[NT - 1]:
        cp.wait()
    out_copies[(NT - 1) % 2] = pltpu.async_copy(
        rows_v.at[(NT - 1) % 2], out_hbm.at[NT - 1, pl.ds(base, RPW)], osem)
    for oc in out_copies:
        if oc is not None:
            oc.wait()


@functools.partial(jax.jit, static_argnums=())
def _sc_gather(idx, emb_test, emb_q, emb_tag):
    mesh = plsc.VectorSubcoreMesh(core_axis_name="c", subcore_axis_name="s")
    return pl.kernel(
        _sc_gather_body,
        out_type=jax.ShapeDtypeStruct((NT, BT, E), jnp.float32),
        mesh=mesh,
        compiler_params=pltpu.CompilerParams(use_tc_tiling_on_sc=False),
        scratch_types=[
            pltpu.VMEM((NT, NCH, CH), jnp.int32),
            pltpu.VMEM((2, RPW, E), jnp.float32),
            pltpu.SemaphoreType.DMA,
            pltpu.SemaphoreType.DMA,
        ],
    )(idx, emb_test, emb_q, emb_tag)


def _tc_body(e_ref, inter_ref, P_ref, Wt_ref, bc_ref,
             Wih0_ref, Whh0_ref, b0_ref,
             Wih1_ref, Whh1_ref, b1_ref, Wf_ref, bf_ref,
             out_ref, Xg_ref, hseq_ref):
    cdims = (((1,), (1,)), ((), ()))

    em = e_ref[...]
    Wt = Wt_ref[...]
    Xp = lax.dot_general(em[0].reshape(T * BB4, FE), Wt[0], cdims,
                         preferred_element_type=jnp.float32)
    for jj in range(1, NT):
        Xp += lax.dot_general(em[jj].reshape(T * BB4, FE), Wt[jj], cdims,
                              preferred_element_type=jnp.float32)
    p0 = P_ref[0:1, :]
    p1 = P_ref[1:2, :]
    p2 = P_ref[2:3, :]
    for k in range(4):
        iv = inter_ref[k]
        pc = jnp.where(iv == 0, p0, jnp.where(iv == 1, p1, p2))
        Xk = Xp[:, k * H:(k + 1) * H] + pc + bc_ref[...]
        Xg_ref[:, k] = (lax.dot_general(Xk, Wih0_ref[...], cdims,
                                        preferred_element_type=jnp.float32)
                        + b0_ref[...]).reshape(T, BB4, G4)

    def recur(Whh_ref):
        def step(t, carry):
            h, c = carry
            g = Xg_ref[t].reshape(BB, G4) + lax.dot_general(
                h, Whh_ref[...], cdims, preferred_element_type=jnp.float32)
            i = jax.nn.sigmoid(g[:, 0:H])
            f = jax.nn.sigmoid(g[:, H:2 * H])
            gg = jnp.tanh(g[:, 2 * H:3 * H])
            o = jax.nn.sigmoid(g[:, 3 * H:4 * H])
            c = f * c + i * gg
            h = o * jnp.tanh(c)
            hseq_ref[t] = h
            return (h, c)
        z = jnp.zeros((BB, H), jnp.float32)
        lax.fori_loop(0, T, step, (z, z))

    recur(Whh0_ref)
    Xg_ref[...] = (lax.dot_general(hseq_ref[...].reshape(T * BB, H),
                                   Wih1_ref[...], cdims,
                                   preferred_element_type=jnp.float32)
                   + b1_ref[...]).reshape(T, 4, BB4, G4)
    recur(Whh1_ref)
    out_ref[...] = (jnp.sum(hseq_ref[...] * Wf_ref[...][0], axis=-1)
                    + bf_ref[0, 0])


def _tc_lstm(e_tm, inter_p, P3, Wt, bc, Wih0, Whh0, b0,
             Wih1, Whh1, b1, Wf, bf, interpret=False):
    return pl.pallas_call(
        _tc_body,
        out_shape=jax.ShapeDtypeStruct((T, BB), jnp.float32),
        scratch_shapes=[
            pltpu.VMEM((T, 4, BB4, G4), jnp.float32),
            pltpu.VMEM((T, BB, H), jnp.float32),
        ],
        interpret=interpret,
    )(e_tm, inter_p, P3, Wt, bc, Wih0, Whh0, b0, Wih1, Whh1, b1, Wf, bf)


def kernel(test, question, tag, correct, mask, interaction, duration,
           emb_inter, emb_test, emb_q, emb_tag, Wc, bc,
           Wih0, Whh0, bih0, bhh0, Wih1, Whh1, bih1, bhh1, Wf, bf):
    Wt = jnp.stack([
        jax.scipy.linalg.block_diag(*([Wc[:, j * E:(j + 1) * E]] * 4))
        for j in range(1, 4)
    ])
    P3 = emb_inter @ Wc[:, 0:E].T
    bc_r = bc.reshape(1, H)
    b0 = (bih0 + bhh0).reshape(1, G4)
    b1 = (bih1 + bhh1).reshape(1, G4)
    bf_r = bf.reshape(1, 1)
    outs = []
    for i in range(NBP):
        sl = slice(i * BB, (i + 1) * BB)
        idx = jnp.stack([
            test[sl].T.reshape(-1), question[sl].T.reshape(-1),
            tag[sl].T.reshape(-1),
        ]).reshape(NT, NW, NCH, CH).transpose(1, 0, 2, 3)
        e = _sc_gather(idx, emb_test, emb_q, emb_tag)
        e_tm = e.reshape(NT, T, BB4, FE)
        inter_p = interaction[sl].reshape(BB4, 4, T).transpose(1, 2, 0)
        inter_p = inter_p.reshape(4, T * BB4, 1)
        out_i = _tc_lstm(e_tm, inter_p, P3, Wt, bc_r, Wih0, Whh0, b0,
                         Wih1, Whh1, b1, Wf, bf_r)
        outs.append(out_i.reshape(T, 4, BB4).transpose(0, 2, 1))
    out_tm = jnp.concatenate(outs, axis=1).reshape(T, B)
    return out_tm.T


# speedup vs baseline: 2.8095x; 1.4225x over previous
import functools

import jax
import jax.numpy as jnp
from jax import lax
from jax.experimental import pallas as pl
from jax.experimental.pallas import tpu as pltpu
from jax.experimental.pallas import tpu_sc as plsc

B, T, H = 1024, 50, 96
E = 32
G4 = 4 * H
FE = 4 * E
B4 = B // 4

CT = 5
NCHK = T // CT

NT = 3
NC, NS = 2, 16
NW = NC * NS
BT = B * T
RPW = BT // NW
CH = 80
NCH = RPW // CH


def _sc_gather_body(idx_hbm, t_test, t_q, t_tag, out_hbm,
                    idx_v, rows_v, gsem, osem):
    wid = lax.axis_index("s") * NC + lax.axis_index("c")
    base = wid * RPW
    tables = (t_test, t_q, t_tag)
    pltpu.sync_copy(idx_hbm.at[wid], idx_v)
    out_copies = [None, None]
    gather_waves = []
    for j, tab in enumerate(tables):
        s = j % 2
        if out_copies[s] is not None:
            out_copies[s].wait()
        copies = []
        for c in range(NCH):
            copies.append(
                pltpu.async_copy(tab.at[idx_v.at[j, c]],
                                 rows_v.at[s, pl.ds(c * CH, CH)], gsem))
        gather_waves.append(copies)
        if j >= 1:
            for cp in gather_waves[j - 1]:
                cp.wait()
            out_copies[(j - 1) % 2] = pltpu.async_copy(
                rows_v.at[(j - 1) % 2],
                out_hbm.at[j - 1, pl.ds(base, RPW)], osem)
    for cp in gather_waves[NT - 1]:
        cp.wait()
    out_copies[(NT - 1) % 2] = pltpu.async_copy(
        rows_v.at[(NT - 1) % 2], out_hbm.at[NT - 1, pl.ds(base, RPW)], osem)
    for oc in out_copies:
        if oc is not None:
            oc.wait()


@functools.partial(jax.jit, static_argnums=())
def _sc_gather(idx, emb_test, emb_q, emb_tag):
    mesh = plsc.VectorSubcoreMesh(core_axis_name="c", subcore_axis_name="s")
    return pl.kernel(
        _sc_gather_body,
        out_type=jax.ShapeDtypeStruct((NT, BT, E), jnp.float32),
        mesh=mesh,
        compiler_params=pltpu.CompilerParams(use_tc_tiling_on_sc=False),
        scratch_types=[
            pltpu.VMEM((NT, NCH, CH), jnp.int32),
            pltpu.VMEM((2, RPW, E), jnp.float32),
            pltpu.SemaphoreType.DMA,
            pltpu.SemaphoreType.DMA,
        ],
    )(idx, emb_test, emb_q, emb_tag)


def _tc_body(e_ref, inter_ref, P_ref, Wt_ref, bc_ref,
             Wih0_ref, Whh0_ref, b0_ref,
             Wih1_ref, Whh1_ref, b1_ref, Wf_ref, bf_ref,
             out_ref, Xg0_ref, h0s_ref, Xg1_ref):
    cdims = (((1,), (1,)), ((), ()))
    Wt = Wt_ref[...]
    p0 = P_ref[0:1, :]
    p1 = P_ref[1:2, :]
    p2 = P_ref[2:3, :]
    z = jnp.zeros((B, H), jnp.float32)
    h0 = c0 = h1 = c1 = z
    wf = Wf_ref[...][0]
    bf = bf_ref[0, 0]

    for c in range(NCHK):
        base = c * CT
        em = e_ref[:, base:base + CT]
        Xp = lax.dot_general(em[0].reshape(CT * B4, FE), Wt[0], cdims,
                             preferred_element_type=jnp.float32)
        for jj in range(1, NT):
            Xp += lax.dot_general(em[jj].reshape(CT * B4, FE), Wt[jj],
                                  cdims, preferred_element_type=jnp.float32)
        for k in range(4):
            iv = inter_ref[k, base * B4:(base + CT) * B4]
            pc = jnp.where(iv == 0, p0, jnp.where(iv == 1, p1, p2))
            Xk = Xp[:, k * H:(k + 1) * H] + pc + bc_ref[...]
            Xg0_ref[:, k] = (
                lax.dot_general(Xk, Wih0_ref[...], cdims,
                                preferred_element_type=jnp.float32)
                + b0_ref[...]).reshape(CT, B4, G4)

        for tt in range(CT):
            g = Xg0_ref[tt].reshape(B, G4) + lax.dot_general(
                h0, Whh0_ref[...], cdims,
                preferred_element_type=jnp.float32)
            i = jax.nn.sigmoid(g[:, 0:H])
            f = jax.nn.sigmoid(g[:, H:2 * H])
            gg = jnp.tanh(g[:, 2 * H:3 * H])
            o = jax.nn.sigmoid(g[:, 3 * H:4 * H])
            c0 = f * c0 + i * gg
            h0 = o * jnp.tanh(c0)
            h0s_ref[tt] = h0

        Xg1_ref[...] = (
            lax.dot_general(h0s_ref[...].reshape(CT * B, H), Wih1_ref[...],
                            cdims, preferred_element_type=jnp.float32)
            + b1_ref[...]).reshape(CT, B, G4)

        for tt in range(CT):
            g = Xg1_ref[tt] + lax.dot_general(
                h1, Whh1_ref[...], cdims,
                preferred_element_type=jnp.float32)
            i = jax.nn.sigmoid(g[:, 0:H])
            f = jax.nn.sigmoid(g[:, H:2 * H])
            gg = jnp.tanh(g[:, 2 * H:3 * H])
            o = jax.nn.sigmoid(g[:, 3 * H:4 * H])
            c1 = f * c1 + i * gg
            h1 = o * jnp.tanh(c1)
            out_ref[base + tt] = jnp.sum(h1 * wf, axis=-1) + bf


def _tc_lstm(e_tm, inter_p, P3, Wt, bc, Wih0, Whh0, b0,
             Wih1, Whh1, b1, Wf, bf, interpret=False):
    return pl.pallas_call(
        _tc_body,
        out_shape=jax.ShapeDtypeStruct((T, B), jnp.float32),
        scratch_shapes=[
            pltpu.VMEM((CT, 4, B4, G4), jnp.float32),
            pltpu.VMEM((CT, B, H), jnp.float32),
            pltpu.VMEM((CT, B, G4), jnp.float32),
        ],
        interpret=interpret,
    )(e_tm, inter_p, P3, Wt, bc, Wih0, Whh0, b0, Wih1, Whh1, b1, Wf, bf)


def kernel(test, question, tag, correct, mask, interaction, duration,
           emb_inter, emb_test, emb_q, emb_tag, Wc, bc,
           Wih0, Whh0, bih0, bhh0, Wih1, Whh1, bih1, bhh1, Wf, bf):
    Wt = jnp.stack([
        jax.scipy.linalg.block_diag(*([Wc[:, j * E:(j + 1) * E]] * 4))
        for j in range(1, 4)
    ])
    P3 = emb_inter @ Wc[:, 0:E].T
    bc_r = bc.reshape(1, H)
    b0 = (bih0 + bhh0).reshape(1, G4)
    b1 = (bih1 + bhh1).reshape(1, G4)
    bf_r = bf.reshape(1, 1)
    idx = jnp.stack([
        test.T.reshape(-1), question.T.reshape(-1), tag.T.reshape(-1),
    ]).reshape(NT, NW, NCH, CH).transpose(1, 0, 2, 3)
    e = _sc_gather(idx, emb_test, emb_q, emb_tag)
    e_tm = e.reshape(NT, T, B4, FE)
    inter_p = interaction.reshape(B4, 4, T).transpose(1, 2, 0)
    inter_p = inter_p.reshape(4, T * B4, 1)
    out_p = _tc_lstm(e_tm, inter_p, P3, Wt, bc_r, Wih0, Whh0, b0,
                     Wih1, Whh1, b1, Wf, bf_r)
    out_tm = out_p.reshape(T, 4, B4).transpose(0, 2, 1).reshape(T, B)
    return out_tm.T
